# speedup vs baseline: 3.3209x; 3.3209x over previous
import functools

import jax
import jax.numpy as jnp
from jax import lax
from jax.experimental import pallas as pl
from jax.experimental.pallas import tpu as pltpu
from jax.experimental.pallas import tpu_sc as plsc

NUM_CORES = 2
NUM_SUBCORES = 16
NUM_WORKERS = NUM_CORES * NUM_SUBCORES
CHUNK = 128
NBUF = 2


@functools.partial(jax.jit, static_argnums=(2, 3))
def _embed_lookup(idx, table, n_chunks, d_model):
    n_rows = NUM_WORKERS * n_chunks * CHUNK
    mesh = plsc.VectorSubcoreMesh(core_axis_name="c", subcore_axis_name="s")

    @functools.partial(
        pl.kernel,
        mesh=mesh,
        out_type=jax.ShapeDtypeStruct((n_rows, d_model), jnp.float32),
        scratch_types=[
            pltpu.VMEM((n_chunks, CHUNK), jnp.int32),
            pltpu.VMEM((NBUF, CHUNK, d_model), jnp.float32),
            pltpu.SemaphoreType.DMA,
            pltpu.SemaphoreType.DMA,
            pltpu.SemaphoreType.DMA,
            pltpu.SemaphoreType.DMA,
        ],
    )
    def k(idx_hbm, table_hbm, out_hbm, idx_v, rows_v, g0, g1, s0, s1):
        wid = lax.axis_index("s") * NUM_CORES + lax.axis_index("c")
        pltpu.sync_copy(idx_hbm.at[wid], idx_v)
        base = wid * (n_chunks * CHUNK)
        gsems = (g0, g1)
        ssems = (s0, s1)

        def start_gather(j, b):
            pltpu.async_copy(table_hbm.at[idx_v.at[j]], rows_v.at[b], gsems[b])

        def wait_gather(b):
            pltpu.make_async_copy(
                table_hbm.at[idx_v.at[0]], rows_v.at[b], gsems[b]
            ).wait()

        def start_store(j, b):
            pltpu.async_copy(
                rows_v.at[b], out_hbm.at[pl.ds(base + j * CHUNK, CHUNK)], ssems[b]
            )

        def wait_store(b):
            pltpu.make_async_copy(
                rows_v.at[b], out_hbm.at[pl.ds(base, CHUNK)], ssems[b]
            ).wait()

        for b in range(NBUF):
            start_gather(b, b)

        def body(i, carry):
            j0 = i * NBUF
            for b in range(NBUF):
                j = j0 + b
                wait_gather(b)
                start_store(j, b)
                wait_store(b)

                @pl.when(j + NBUF < n_chunks)
                def _():
                    start_gather(j + NBUF, b)

            return carry

        lax.fori_loop(0, n_chunks // NBUF, body, 0)

    return k(idx, table)


def kernel(labels, label_embed_weight):
    b, s = labels.shape
    v, d = label_embed_weight.shape
    n_chunks = (b * s) // (NUM_WORKERS * CHUNK)
    idx = labels.astype(jnp.int32).reshape(NUM_WORKERS, n_chunks, CHUNK)
    out = _embed_lookup(idx, label_embed_weight, n_chunks, d)
    return out.reshape(b, s, d)
